# baseline (speedup 1.0000x reference)
import functools

import jax
import jax.numpy as jnp
from jax import lax
from jax.experimental import pallas as pl
from jax.experimental.pallas import tpu as pltpu
from jax.experimental.pallas import tpu_sc as plsc

_N = 10000
_E = 320000
_D = 128
_ED = 16
_H = 64
_NL = 4
_NP = 10240
_NP2 = _NP // 2

_NC = 2
_NS = 16
_NW = _NC * _NS
_EPW = _E // _NW
_CHUNK = 80
_NCHUNK = _EPW // _CHUNK
_NPS = _N // _NS

_mesh = plsc.VectorSubcoreMesh(
    core_axis_name="c", subcore_axis_name="s", num_cores=_NC, num_subcores=_NS
)


_SB = 5
_SUP = _SB * _CHUNK
_NSUP = _EPW // _SUP


def _issue_gathers(p_hbm, idx_v, buf, sem, sup):
    for k in range(_SB):
        pltpu.async_copy(
            p_hbm.at[idx_v.at[sup * _SB + k]], buf.at[pl.ds(k * _CHUNK, _CHUNK)], sem
        )


def _drain_gathers(p_hbm, idx_v, buf, sem, sup):
    for k in range(_SB):
        pltpu.make_async_copy(
            p_hbm.at[idx_v.at[sup * _SB + k]], buf.at[pl.ds(k * _CHUNK, _CHUNK)], sem
        ).wait()


@functools.partial(
    pl.kernel,
    out_type=jax.ShapeDtypeStruct((_E, _H), jnp.bfloat16),
    mesh=_mesh,
    scratch_types=[
        pltpu.VMEM((_NCHUNK, _CHUNK), jnp.int32),
        pltpu.VMEM((_SUP, _H), jnp.bfloat16),
        pltpu.VMEM((_SUP, _H), jnp.bfloat16),
        pltpu.VMEM((_SUP, _H), jnp.bfloat16),
        pltpu.SemaphoreType.DMA,
        pltpu.SemaphoreType.DMA,
        pltpu.SemaphoreType.DMA,
        pltpu.SemaphoreType.DMA,
        pltpu.SemaphoreType.DMA,
        pltpu.SemaphoreType.DMA,
    ],
    compiler_params=pltpu.CompilerParams(use_tc_tiling_on_sc=False),
)
def _sc_gather(p_hbm, src_hbm, out_hbm, idx_v, b0, b1, b2, g0, g1, g2, s0, s1, s2):
    wid = lax.axis_index("c") * _NS + lax.axis_index("s")
    base = wid * _EPW
    bufs = (b0, b1, b2)
    gsem = (g0, g1, g2)
    ssem = (s0, s1, s2)
    pltpu.sync_copy(src_hbm.at[wid], idx_v)
    _issue_gathers(p_hbm, idx_v, b0, g0, 0)

    def _out_slice(sup):
        return out_hbm.at[pl.ds(base + sup * _SUP, _SUP)]

    def _step(i, j):
        jn = (j + 1) % 3

        @pl.when(i >= 2)
        def _():
            pltpu.make_async_copy(bufs[jn], _out_slice(i - 2), ssem[jn]).wait()

        @pl.when(i + 1 < _NSUP)
        def _():
            _issue_gathers(p_hbm, idx_v, bufs[jn], gsem[jn], i + 1)

        _drain_gathers(p_hbm, idx_v, bufs[j], gsem[j], i)
        pltpu.async_copy(bufs[j], _out_slice(i), ssem[j])

    def body(g, carry):
        for j in range(3):
            _step(3 * g + j, j)
        return carry

    lax.fori_loop(0, _NSUP // 3, body, 0)
    _step(_NSUP - 1, (_NSUP - 1) % 3)
    for i in range(_NSUP - 2, _NSUP):
        pltpu.make_async_copy(bufs[i % 3], _out_slice(i), ssem[i % 3]).wait()


@functools.partial(
    pl.kernel,
    out_type=jax.ShapeDtypeStruct((_NC, _N, _H), jnp.float32),
    mesh=_mesh,
    scratch_types=[
        pltpu.VMEM((_NCHUNK, _CHUNK), jnp.int32),
        pltpu.VMEM((_SUP, _H), jnp.float32),
        pltpu.VMEM((_SUP, _H), jnp.float32),
        pltpu.VMEM((_SUP, _H), jnp.float32),
        pltpu.VMEM((_NPS // 25, _H), jnp.float32),
        pltpu.VMEM_SHARED((_N, _H), jnp.float32),
        pltpu.SemaphoreType.DMA,
        pltpu.SemaphoreType.DMA,
        pltpu.SemaphoreType.DMA,
        pltpu.SemaphoreType.DMA,
        pltpu.SemaphoreType.DMA,
        pltpu.SemaphoreType.DMA,
    ],
    compiler_params=pltpu.CompilerParams(use_tc_tiling_on_sc=False),
)
def _sc_scatter(
    m_hbm, dst_hbm, out_hbm, idx_v, b0, b1, b2, zbuf, acc_sh, l0, l1, l2, a0, a1, a2
):
    cc = lax.axis_index("c")
    s = lax.axis_index("s")
    wid = cc * _NS + s
    base = wid * _EPW
    bufs = (b0, b1, b2)
    lsem = (l0, l1, l2)
    asem = (a0, a1, a2)

    def _m_slice(sup):
        return m_hbm.at[pl.ds(base + sup * _SUP, _SUP)]

    def _issue_adds(buf, sem, sup):
        for k in range(_SB):
            pltpu.async_copy(
                buf.at[pl.ds(k * _CHUNK, _CHUNK)],
                acc_sh.at[idx_v.at[sup * _SB + k]],
                sem,
                add=True,
            )

    def _drain_adds(buf, sem, sup):
        for k in range(_SB):
            pltpu.make_async_copy(
                buf.at[pl.ds(k * _CHUNK, _CHUNK)],
                acc_sh.at[idx_v.at[sup * _SB + k]],
                sem,
            ).wait()

    def zb(k, carry):
        zbuf[k // 4, pl.ds((k % 4) * 16, 16)] = jnp.zeros((16,), jnp.float32)
        return carry

    _ZR = _NPS // 25
    lax.fori_loop(0, _ZR * 4, zb, 0)

    def zcp(r, carry):
        pltpu.sync_copy(zbuf, acc_sh.at[pl.ds(s * _NPS + r * _ZR, _ZR)])
        return carry

    lax.fori_loop(0, _NPS // _ZR, zcp, 0)
    pltpu.sync_copy(dst_hbm.at[wid], idx_v)
    pltpu.async_copy(_m_slice(0), b0, l0)
    plsc.subcore_barrier()

    def _step(i, j):
        jn = (j + 1) % 3

        @pl.when(i >= 2)
        def _():
            _drain_adds(bufs[jn], asem[jn], i - 2)

        @pl.when(i + 1 < _NSUP)
        def _():
            pltpu.async_copy(_m_slice(i + 1), bufs[jn], lsem[jn])

        pltpu.make_async_copy(_m_slice(i), bufs[j], lsem[j]).wait()
        _issue_adds(bufs[j], asem[j], i)

    def body(g, carry):
        for j in range(3):
            _step(3 * g + j, j)
        return carry

    lax.fori_loop(0, _NSUP // 3, body, 0)
    _step(_NSUP - 1, (_NSUP - 1) % 3)
    for i in range(_NSUP - 2, _NSUP):
        _drain_adds(bufs[i % 3], asem[i % 3], i)
    plsc.subcore_barrier()
    pltpu.sync_copy(
        acc_sh.at[pl.ds(s * _NPS, _NPS)], out_hbm.at[cc, pl.ds(s * _NPS, _NPS)]
    )


_N2 = _N // 2


def _embed_body(x_ref, ewT_ref, eb_ref, w1hT_ref, b1_ref, h_ref, p_ref):
    h = jax.nn.silu(
        jnp.dot(x_ref[...], ewT_ref[...], preferred_element_type=jnp.float32)
        + eb_ref[...]
    )
    h_ref[...] = h
    p_ref[pl.ds(0, _N2), :] = (
        jnp.dot(h, w1hT_ref[...], preferred_element_type=jnp.float32) + b1_ref[...]
    ).astype(jnp.bfloat16)


_embed = pl.pallas_call(
    _embed_body,
    out_shape=[
        jax.ShapeDtypeStruct((_N2, 2 * _H), jnp.float32),
        jax.ShapeDtypeStruct((_NP2, 2 * _H), jnp.bfloat16),
    ],
)

_BE = 1600


def _msg_body(g_ref, ea_ref, w1eT_ref, w2T_ref, b2_ref, out_ref):
    m1 = jax.nn.silu(
        g_ref[...].astype(jnp.float32)
        + jnp.dot(ea_ref[...], w1eT_ref[...], preferred_element_type=jnp.float32)
    )
    out_ref[...] = jax.nn.silu(
        jnp.dot(m1, w2T_ref[...], preferred_element_type=jnp.float32) + b2_ref[...]
    )


_msg = pl.pallas_call(
    _msg_body,
    grid=(_E // 2 // _BE,),
    in_specs=[
        pl.BlockSpec((_BE, 2 * _H), lambda i: (i, 0)),
        pl.BlockSpec((_BE, 2 * _ED), lambda i: (i, 0)),
        pl.BlockSpec((2 * _ED, 2 * _H), lambda i: (0, 0)),
        pl.BlockSpec((2 * _H, 2 * _H), lambda i: (0, 0)),
        pl.BlockSpec((1, 2 * _H), lambda i: (0, 0)),
    ],
    out_specs=pl.BlockSpec((_BE, 2 * _H), lambda i: (i, 0)),
    out_shape=jax.ShapeDtypeStruct((_E // 2, 2 * _H), jnp.float32),
)


def _blockdiag2(w):
    z = jnp.zeros_like(w)
    return jnp.concatenate(
        [jnp.concatenate([w, z], axis=1), jnp.concatenate([z, w], axis=1)], axis=0
    )


def _gru_w2(W):
    WT = W.T
    return jnp.concatenate(
        [_blockdiag2(WT[:, k * _H : (k + 1) * _H]) for k in range(3)], axis=1
    )


def _gru_b2(b):
    return jnp.concatenate(
        [jnp.tile(b[k * _H : (k + 1) * _H], 2) for k in range(3)]
    )[None]


def _gru_core(h, a0_ref, a1_ref, wihT_ref, bih_ref, whhT_ref, bhh_ref):
    agg = a0_ref[...] + a1_ref[...]
    gi = (
        jnp.dot(agg, wihT_ref[...], preferred_element_type=jnp.float32)
        + bih_ref[...]
    )
    gh = jnp.dot(h, whhT_ref[...], preferred_element_type=jnp.float32) + bhh_ref[...]
    hp = 2 * _H
    r = jax.nn.sigmoid(gi[:, :hp] + gh[:, :hp])
    z = jax.nn.sigmoid(gi[:, hp : 2 * hp] + gh[:, hp : 2 * hp])
    n = jnp.tanh(gi[:, 2 * hp :] + r * gh[:, 2 * hp :])
    return (1.0 - z) * n + z * h


def _gru_body(
    h_ref, a0_ref, a1_ref, wihT_ref, bih_ref, whhT_ref, bhh_ref, w1hT_ref, b1_ref,
    hout_ref, pout_ref,
):
    hn = _gru_core(h_ref[...], a0_ref, a1_ref, wihT_ref, bih_ref, whhT_ref, bhh_ref)
    hout_ref[...] = hn
    pout_ref[pl.ds(0, _N2), :] = (
        jnp.dot(hn, w1hT_ref[...], preferred_element_type=jnp.float32) + b1_ref[...]
    ).astype(jnp.bfloat16)


_gru = pl.pallas_call(
    _gru_body,
    out_shape=[
        jax.ShapeDtypeStruct((_N2, 2 * _H), jnp.float32),
        jax.ShapeDtypeStruct((_NP2, 2 * _H), jnp.bfloat16),
    ],
)


def _gru_head_body(
    h_ref, a0_ref, a1_ref, wihT_ref, bih_ref, whhT_ref, bhh_ref, hw1T_ref, hb1_ref,
    hw2_ref, hb2_ref, out_ref,
):
    hn = _gru_core(h_ref[...], a0_ref, a1_ref, wihT_ref, bih_ref, whhT_ref, bhh_ref)
    sh = jax.nn.silu(
        jnp.dot(hn, hw1T_ref[...], preferred_element_type=jnp.float32) + hb1_ref[...]
    )
    v = jnp.sum(sh, axis=0, keepdims=True)
    total = jnp.sum(v * hw2_ref[...]) + _N * hb2_ref[0, 0]
    out_ref[...] = jnp.reshape(total, (1, 1))


_gru_head = pl.pallas_call(
    _gru_head_body,
    out_shape=jax.ShapeDtypeStruct((1, 1), jnp.float32),
)


def kernel(
    x, edge_index, edge_attr, embed_W, embed_b, msg_W1, msg_b1, msg_W2, msg_b2,
    gru_Wih, gru_bih, gru_Whh, gru_bhh, head_W1, head_b1, head_W2, head_b2,
):
    src = edge_index[0].reshape(_NW, _NCHUNK, _CHUNK)
    dst = edge_index[1].reshape(_NW, _NCHUNK, _CHUNK)
    h2, p2 = _embed(
        x.reshape(_N2, 2 * _D),
        _blockdiag2(embed_W.T),
        jnp.tile(embed_b, 2)[None],
        _blockdiag2(msg_W1[0, :, :_H].T),
        jnp.tile(msg_b1[0], 2)[None],
    )
    ea2 = edge_attr.reshape(_E // 2, 2 * _ED)
    out = None
    for l in range(_NL):
        g = _sc_gather(p2.reshape(_NP, _H), src)
        m22 = _msg(
            g.reshape(_E // 2, 2 * _H),
            ea2,
            _blockdiag2(msg_W1[l, :, _H:].T),
            _blockdiag2(msg_W2[l].T),
            jnp.tile(msg_b2[l], 2)[None],
        )
        aggp = _sc_scatter(m22.reshape(_E, _H), dst)
        a2 = aggp.reshape(_NC, _N2, 2 * _H)
        if l < _NL - 1:
            h2, p2 = _gru(
                h2, a2[0], a2[1],
                _gru_w2(gru_Wih[l]), _gru_b2(gru_bih[l]),
                _gru_w2(gru_Whh[l]), _gru_b2(gru_bhh[l]),
                _blockdiag2(msg_W1[l + 1, :, :_H].T),
                jnp.tile(msg_b1[l + 1], 2)[None],
            )
        else:
            out = _gru_head(
                h2, a2[0], a2[1],
                _gru_w2(gru_Wih[l]), _gru_b2(gru_bih[l]),
                _gru_w2(gru_Whh[l]), _gru_b2(gru_bhh[l]),
                _blockdiag2(head_W1.T), jnp.tile(head_b1, 2)[None],
                jnp.tile(head_W2[0], 2)[None], head_b2.reshape(1, 1),
            )
    return out.reshape((1,))


# speedup vs baseline: 1.4421x; 1.4421x over previous
import functools

import jax
import jax.numpy as jnp
from jax import lax
from jax.experimental import pallas as pl
from jax.experimental.pallas import tpu as pltpu
from jax.experimental.pallas import tpu_sc as plsc

_N = 10000
_E = 320000
_D = 128
_ED = 16
_H = 64
_NL = 4
_N2 = _N // 2

_NC = 2
_NS = 16
_NW = _NC * _NS
_EH = _E // 2
_EPW = _EH // _NW
_CHUNK = 40
_NCHUNK = _EPW // _CHUNK
_NPS = _N // _NS
_SB = 5
_SUP = _SB * _CHUNK
_NSUP = _EPW // _SUP

_mesh = plsc.VectorSubcoreMesh(
    core_axis_name="c", subcore_axis_name="s", num_cores=_NC, num_subcores=_NS
)


def _issue_gathers(p_hbm, idx_v, buf, sem, sup):
    for k in range(_SB):
        pltpu.async_copy(
            p_hbm.at[idx_v.at[sup * _SB + k]], buf.at[pl.ds(k * _CHUNK, _CHUNK)], sem
        )


def _drain_gathers(p_hbm, idx_v, buf, sem, sup):
    for k in range(_SB):
        pltpu.make_async_copy(
            p_hbm.at[idx_v.at[sup * _SB + k]], buf.at[pl.ds(k * _CHUNK, _CHUNK)], sem
        ).wait()


@functools.partial(
    pl.kernel,
    out_type=jax.ShapeDtypeStruct((_EH, _H), jnp.float32),
    mesh=_mesh,
    scratch_types=[
        pltpu.VMEM((_NCHUNK, _CHUNK), jnp.int32),
        pltpu.VMEM((_SUP, _H), jnp.float32),
        pltpu.VMEM((_SUP, _H), jnp.float32),
        pltpu.VMEM((_SUP, _H), jnp.float32),
        pltpu.SemaphoreType.DMA,
        pltpu.SemaphoreType.DMA,
        pltpu.SemaphoreType.DMA,
        pltpu.SemaphoreType.DMA,
        pltpu.SemaphoreType.DMA,
        pltpu.SemaphoreType.DMA,
    ],
    compiler_params=pltpu.CompilerParams(use_tc_tiling_on_sc=False),
)
def _sc_gather(p_hbm, src_hbm, out_hbm, idx_v, b0, b1, b2, g0, g1, g2, s0, s1, s2):
    wid = lax.axis_index("c") * _NS + lax.axis_index("s")
    base = wid * _EPW
    bufs = (b0, b1, b2)
    gsem = (g0, g1, g2)
    ssem = (s0, s1, s2)
    pltpu.sync_copy(src_hbm.at[wid], idx_v)
    _issue_gathers(p_hbm, idx_v, b0, g0, 0)

    def _out_slice(sup):
        return out_hbm.at[pl.ds(base + sup * _SUP, _SUP)]

    def _step(i, j):
        jn = (j + 1) % 3

        @pl.when(i >= 2)
        def _():
            pltpu.make_async_copy(bufs[jn], _out_slice(i - 2), ssem[jn]).wait()

        @pl.when(i + 1 < _NSUP)
        def _():
            _issue_gathers(p_hbm, idx_v, bufs[jn], gsem[jn], i + 1)

        _drain_gathers(p_hbm, idx_v, bufs[j], gsem[j], i)
        pltpu.async_copy(bufs[j], _out_slice(i), ssem[j])

    def body(g, carry):
        for j in range(3):
            _step(3 * g + j, j)
        return carry

    lax.fori_loop(0, _NSUP // 3, body, 0)
    _step(_NSUP - 1, (_NSUP - 1) % 3)
    for i in range(_NSUP - 2, _NSUP):
        pltpu.make_async_copy(bufs[i % 3], _out_slice(i), ssem[i % 3]).wait()


@functools.partial(
    pl.kernel,
    out_type=jax.ShapeDtypeStruct((_NC, _N, _H), jnp.float32),
    mesh=_mesh,
    scratch_types=[
        pltpu.VMEM((_NCHUNK, _CHUNK), jnp.int32),
        pltpu.VMEM((_SUP, _H), jnp.float32),
        pltpu.VMEM((_SUP, _H), jnp.float32),
        pltpu.VMEM((_SUP, _H), jnp.float32),
        pltpu.VMEM((_NPS // 25, _H), jnp.float32),
        pltpu.VMEM_SHARED((_N, _H), jnp.float32),
        pltpu.SemaphoreType.DMA,
        pltpu.SemaphoreType.DMA,
        pltpu.SemaphoreType.DMA,
        pltpu.SemaphoreType.DMA,
        pltpu.SemaphoreType.DMA,
        pltpu.SemaphoreType.DMA,
    ],
    compiler_params=pltpu.CompilerParams(use_tc_tiling_on_sc=False),
)
def _sc_scatter(
    m_hbm, dst_hbm, out_hbm, idx_v, b0, b1, b2, zbuf, acc_sh, l0, l1, l2, a0, a1, a2
):
    cc = lax.axis_index("c")
    s = lax.axis_index("s")
    wid = cc * _NS + s
    base = wid * _EPW
    bufs = (b0, b1, b2)
    lsem = (l0, l1, l2)
    asem = (a0, a1, a2)

    def _m_slice(sup):
        return m_hbm.at[pl.ds(base + sup * _SUP, _SUP)]

    def _issue_adds(buf, sem, sup):
        for k in range(_SB):
            pltpu.async_copy(
                buf.at[pl.ds(k * _CHUNK, _CHUNK)],
                acc_sh.at[idx_v.at[sup * _SB + k]],
                sem,
                add=True,
            )

    def _drain_adds(buf, sem, sup):
        for k in range(_SB):
            pltpu.make_async_copy(
                buf.at[pl.ds(k * _CHUNK, _CHUNK)],
                acc_sh.at[idx_v.at[sup * _SB + k]],
                sem,
            ).wait()

    def zb(k, carry):
        zbuf[k // 4, pl.ds((k % 4) * 16, 16)] = jnp.zeros((16,), jnp.float32)
        return carry

    _ZR = _NPS // 25
    lax.fori_loop(0, _ZR * 4, zb, 0)

    def zcp(r, carry):
        pltpu.sync_copy(zbuf, acc_sh.at[pl.ds(s * _NPS + r * _ZR, _ZR)])
        return carry

    lax.fori_loop(0, _NPS // _ZR, zcp, 0)
    pltpu.sync_copy(dst_hbm.at[wid], idx_v)
    pltpu.async_copy(_m_slice(0), b0, l0)
    plsc.subcore_barrier()

    def _step(i, j):
        jn = (j + 1) % 3

        @pl.when(i >= 2)
        def _():
            _drain_adds(bufs[jn], asem[jn], i - 2)

        @pl.when(i + 1 < _NSUP)
        def _():
            pltpu.async_copy(_m_slice(i + 1), bufs[jn], lsem[jn])

        pltpu.make_async_copy(_m_slice(i), bufs[j], lsem[j]).wait()
        _issue_adds(bufs[j], asem[j], i)

    def body(g, carry):
        for j in range(3):
            _step(3 * g + j, j)
        return carry

    lax.fori_loop(0, _NSUP // 3, body, 0)
    _step(_NSUP - 1, (_NSUP - 1) % 3)
    for i in range(_NSUP - 2, _NSUP):
        _drain_adds(bufs[i % 3], asem[i % 3], i)
    plsc.subcore_barrier()
    pltpu.sync_copy(
        acc_sh.at[pl.ds(s * _NPS, _NPS)], out_hbm.at[cc, pl.ds(s * _NPS, _NPS)]
    )


def _embed_body(x_ref, ewT_ref, eb_ref, w1hT_ref, b1_ref, h_ref, p_ref):
    h = jax.nn.silu(
        jnp.dot(x_ref[...], ewT_ref[...], preferred_element_type=jnp.float32)
        + eb_ref[...]
    )
    h_ref[...] = h
    p_ref[...] = (
        jnp.dot(h, w1hT_ref[...], preferred_element_type=jnp.float32) + b1_ref[...]
    )


_embed = pl.pallas_call(
    _embed_body,
    out_shape=[
        jax.ShapeDtypeStruct((_N2, 2 * _H), jnp.float32),
        jax.ShapeDtypeStruct((_N2, 2 * _H), jnp.float32),
    ],
)

_BE = 1600


def _msg_body(g_ref, ea_ref, w1eT_ref, w2T_ref, b2_ref, out_ref):
    m1 = jax.nn.silu(
        g_ref[...]
        + jnp.dot(ea_ref[...], w1eT_ref[...], preferred_element_type=jnp.float32)
    )
    out_ref[...] = jax.nn.silu(
        jnp.dot(m1, w2T_ref[...], preferred_element_type=jnp.float32) + b2_ref[...]
    )


_msg = pl.pallas_call(
    _msg_body,
    grid=(_EH // 2 // _BE,),
    in_specs=[
        pl.BlockSpec((_BE, 2 * _H), lambda i: (i, 0)),
        pl.BlockSpec((_BE, 2 * _ED), lambda i: (i, 0)),
        pl.BlockSpec((2 * _ED, 2 * _H), lambda i: (0, 0)),
        pl.BlockSpec((2 * _H, 2 * _H), lambda i: (0, 0)),
        pl.BlockSpec((1, 2 * _H), lambda i: (0, 0)),
    ],
    out_specs=pl.BlockSpec((_BE, 2 * _H), lambda i: (i, 0)),
    out_shape=jax.ShapeDtypeStruct((_EH // 2, 2 * _H), jnp.float32),
)


def _blockdiag2(w):
    z = jnp.zeros_like(w)
    return jnp.concatenate(
        [jnp.concatenate([w, z], axis=1), jnp.concatenate([z, w], axis=1)], axis=0
    )


def _gru_w2(W):
    WT = W.T
    return jnp.concatenate(
        [_blockdiag2(WT[:, k * _H : (k + 1) * _H]) for k in range(3)], axis=1
    )


def _gru_b2(b):
    return jnp.concatenate(
        [jnp.tile(b[k * _H : (k + 1) * _H], 2) for k in range(3)]
    )[None]


def _gru_core(h, a_refs, wihT_ref, bih_ref, whhT_ref, bhh_ref):
    agg = a_refs[0][...] + a_refs[1][...] + a_refs[2][...] + a_refs[3][...]
    gi = (
        jnp.dot(agg, wihT_ref[...], preferred_element_type=jnp.float32)
        + bih_ref[...]
    )
    gh = jnp.dot(h, whhT_ref[...], preferred_element_type=jnp.float32) + bhh_ref[...]
    hp = 2 * _H
    r = jax.nn.sigmoid(gi[:, :hp] + gh[:, :hp])
    z = jax.nn.sigmoid(gi[:, hp : 2 * hp] + gh[:, hp : 2 * hp])
    n = jnp.tanh(gi[:, 2 * hp :] + r * gh[:, 2 * hp :])
    return (1.0 - z) * n + z * h


def _gru_body(
    h_ref, a0_ref, a1_ref, a2_ref, a3_ref, wihT_ref, bih_ref, whhT_ref, bhh_ref,
    w1hT_ref, b1_ref, hout_ref, pout_ref,
):
    hn = _gru_core(
        h_ref[...], (a0_ref, a1_ref, a2_ref, a3_ref), wihT_ref, bih_ref, whhT_ref,
        bhh_ref,
    )
    hout_ref[...] = hn
    pout_ref[...] = (
        jnp.dot(hn, w1hT_ref[...], preferred_element_type=jnp.float32) + b1_ref[...]
    )


_gru = pl.pallas_call(
    _gru_body,
    out_shape=[
        jax.ShapeDtypeStruct((_N2, 2 * _H), jnp.float32),
        jax.ShapeDtypeStruct((_N2, 2 * _H), jnp.float32),
    ],
)


def _gru_head_body(
    h_ref, a0_ref, a1_ref, a2_ref, a3_ref, wihT_ref, bih_ref, whhT_ref, bhh_ref,
    hw1T_ref, hb1_ref, hw2_ref, hb2_ref, out_ref,
):
    hn = _gru_core(
        h_ref[...], (a0_ref, a1_ref, a2_ref, a3_ref), wihT_ref, bih_ref, whhT_ref,
        bhh_ref,
    )
    sh = jax.nn.silu(
        jnp.dot(hn, hw1T_ref[...], preferred_element_type=jnp.float32) + hb1_ref[...]
    )
    v = jnp.sum(sh, axis=0, keepdims=True)
    total = jnp.sum(v * hw2_ref[...]) + _N * hb2_ref[0, 0]
    out_ref[...] = jnp.reshape(total, (1, 1))


_gru_head = pl.pallas_call(
    _gru_head_body,
    out_shape=jax.ShapeDtypeStruct((1, 1), jnp.float32),
)


def kernel(
    x, edge_index, edge_attr, embed_W, embed_b, msg_W1, msg_b1, msg_W2, msg_b2,
    gru_Wih, gru_bih, gru_Whh, gru_bhh, head_W1, head_b1, head_W2, head_b2,
):
    srcs, dsts = [], []
    for half in range(2):
        sl = slice(half * _EH, (half + 1) * _EH)
        srcs.append(edge_index[0, sl].reshape(_NW, _NCHUNK, _CHUNK))
        dsts.append(edge_index[1, sl].reshape(_NW, _NCHUNK, _CHUNK))
    ea2 = edge_attr.reshape(_E // 2, 2 * _ED)
    eas = [ea2[: _EH // 2], ea2[_EH // 2 :]]
    h2, p2 = _embed(
        x.reshape(_N2, 2 * _D),
        _blockdiag2(embed_W.T),
        jnp.tile(embed_b, 2)[None],
        _blockdiag2(msg_W1[0, :, :_H].T),
        jnp.tile(msg_b1[0], 2)[None],
    )
    out = None
    for l in range(_NL):
        w1e2 = _blockdiag2(msg_W1[l, :, _H:].T)
        w22 = _blockdiag2(msg_W2[l].T)
        b22 = jnp.tile(msg_b2[l], 2)[None]
        p64 = p2.reshape(_N, _H)
        aggs = []
        for half in range(2):
            g = _sc_gather(p64, srcs[half])
            m22 = _msg(g.reshape(_EH // 2, 2 * _H), eas[half], w1e2, w22, b22)
            aggp = _sc_scatter(m22.reshape(_EH, _H), dsts[half])
            a2 = aggp.reshape(_NC, _N2, 2 * _H)
            aggs += [a2[0], a2[1]]
        if l < _NL - 1:
            h2, p2 = _gru(
                h2, *aggs,
                _gru_w2(gru_Wih[l]), _gru_b2(gru_bih[l]),
                _gru_w2(gru_Whh[l]), _gru_b2(gru_bhh[l]),
                _blockdiag2(msg_W1[l + 1, :, :_H].T),
                jnp.tile(msg_b1[l + 1], 2)[None],
            )
        else:
            out = _gru_head(
                h2, *aggs,
                _gru_w2(gru_Wih[l]), _gru_b2(gru_bih[l]),
                _gru_w2(gru_Whh[l]), _gru_b2(gru_bhh[l]),
                _blockdiag2(head_W1.T), jnp.tile(head_b1, 2)[None],
                jnp.tile(head_W2[0], 2)[None], head_b2.reshape(1, 1),
            )
    return out.reshape((1,))


# speedup vs baseline: 1.6410x; 1.1379x over previous
import functools

import jax
import jax.numpy as jnp
from jax import lax
from jax.experimental import pallas as pl
from jax.experimental.pallas import tpu as pltpu
from jax.experimental.pallas import tpu_sc as plsc

_N = 10000
_E = 320000
_D = 128
_ED = 16
_H = 64
_NL = 4
_N2 = _N // 2

_NC = 2
_NS = 16
_NW = _NC * _NS
_EPW = _E // _NW
_CHUNK = 80
_NCHUNK = _EPW // _CHUNK
_NPS = _N // _NS
_SB = 5
_SUP = _SB * _CHUNK
_NSUP = _EPW // _SUP

_mesh = plsc.VectorSubcoreMesh(
    core_axis_name="c", subcore_axis_name="s", num_cores=_NC, num_subcores=_NS
)


def _issue_gathers(p_hbm, idx_v, buf, sem, sup):
    for k in range(_SB):
        pltpu.async_copy(
            p_hbm.at[idx_v.at[sup * _SB + k]], buf.at[pl.ds(k * _CHUNK, _CHUNK)], sem
        )


def _drain_gathers(p_hbm, idx_v, buf, sem, sup):
    for k in range(_SB):
        pltpu.make_async_copy(
            p_hbm.at[idx_v.at[sup * _SB + k]], buf.at[pl.ds(k * _CHUNK, _CHUNK)], sem
        ).wait()


@functools.partial(
    pl.kernel,
    out_type=jax.ShapeDtypeStruct((_E, _H), jnp.float32),
    mesh=_mesh,
    scratch_types=[
        pltpu.VMEM((_NCHUNK, _CHUNK), jnp.int32),
        pltpu.VMEM((_SUP, _H), jnp.float32),
        pltpu.VMEM((_SUP, _H), jnp.float32),
        pltpu.VMEM((_SUP, _H), jnp.float32),
        pltpu.SemaphoreType.DMA,
        pltpu.SemaphoreType.DMA,
        pltpu.SemaphoreType.DMA,
        pltpu.SemaphoreType.DMA,
        pltpu.SemaphoreType.DMA,
        pltpu.SemaphoreType.DMA,
    ],
    compiler_params=pltpu.CompilerParams(use_tc_tiling_on_sc=False),
)
def _sc_gather(p_hbm, src_hbm, out_hbm, idx_v, b0, b1, b2, g0, g1, g2, s0, s1, s2):
    wid = lax.axis_index("c") * _NS + lax.axis_index("s")
    base = wid * _EPW
    bufs = (b0, b1, b2)
    gsem = (g0, g1, g2)
    ssem = (s0, s1, s2)
    pltpu.sync_copy(src_hbm.at[wid], idx_v)
    _issue_gathers(p_hbm, idx_v, b0, g0, 0)

    def _out_slice(sup):
        return out_hbm.at[pl.ds(base + sup * _SUP, _SUP)]

    def _step(i, j):
        jn = (j + 1) % 3

        @pl.when(i >= 2)
        def _():
            pltpu.make_async_copy(bufs[jn], _out_slice(i - 2), ssem[jn]).wait()

        @pl.when(i + 1 < _NSUP)
        def _():
            _issue_gathers(p_hbm, idx_v, bufs[jn], gsem[jn], i + 1)

        _drain_gathers(p_hbm, idx_v, bufs[j], gsem[j], i)
        pltpu.async_copy(bufs[j], _out_slice(i), ssem[j])

    def body(g, carry):
        for j in range(3):
            _step(3 * g + j, j)
        return carry

    lax.fori_loop(0, _NSUP // 3, body, 0)
    _step(_NSUP - 1, (_NSUP - 1) % 3)
    for i in range(_NSUP - 2, _NSUP):
        pltpu.make_async_copy(bufs[i % 3], _out_slice(i), ssem[i % 3]).wait()


@functools.partial(
    pl.kernel,
    out_type=jax.ShapeDtypeStruct((_NC, _N, _H), jnp.float32),
    mesh=_mesh,
    scratch_types=[
        pltpu.VMEM((_NCHUNK, _CHUNK), jnp.int32),
        pltpu.VMEM((_SUP, _H), jnp.float32),
        pltpu.VMEM((_SUP, _H), jnp.float32),
        pltpu.VMEM((_SUP, _H), jnp.float32),
        pltpu.VMEM((_NPS // 25, _H), jnp.float32),
        pltpu.VMEM_SHARED((_N, _H), jnp.float32),
        pltpu.SemaphoreType.DMA,
        pltpu.SemaphoreType.DMA,
        pltpu.SemaphoreType.DMA,
        pltpu.SemaphoreType.DMA,
        pltpu.SemaphoreType.DMA,
        pltpu.SemaphoreType.DMA,
    ],
    compiler_params=pltpu.CompilerParams(use_tc_tiling_on_sc=False),
)
def _sc_scatter(
    m_hbm, dst_hbm, out_hbm, idx_v, b0, b1, b2, zbuf, acc_sh, l0, l1, l2, a0, a1, a2
):
    cc = lax.axis_index("c")
    s = lax.axis_index("s")
    wid = cc * _NS + s
    base = wid * _EPW
    bufs = (b0, b1, b2)
    lsem = (l0, l1, l2)
    asem = (a0, a1, a2)

    def _m_slice(sup):
        return m_hbm.at[pl.ds(base + sup * _SUP, _SUP)]

    def _issue_adds(buf, sem, sup):
        for k in range(_SB):
            pltpu.async_copy(
                buf.at[pl.ds(k * _CHUNK, _CHUNK)],
                acc_sh.at[idx_v.at[sup * _SB + k]],
                sem,
                add=True,
            )

    def _drain_adds(buf, sem, sup):
        for k in range(_SB):
            pltpu.make_async_copy(
                buf.at[pl.ds(k * _CHUNK, _CHUNK)],
                acc_sh.at[idx_v.at[sup * _SB + k]],
                sem,
            ).wait()

    def zb(k, carry):
        zbuf[k // 4, pl.ds((k % 4) * 16, 16)] = jnp.zeros((16,), jnp.float32)
        return carry

    _ZR = _NPS // 25
    lax.fori_loop(0, _ZR * 4, zb, 0)

    def zcp(r, carry):
        pltpu.sync_copy(zbuf, acc_sh.at[pl.ds(s * _NPS + r * _ZR, _ZR)])
        return carry

    lax.fori_loop(0, _NPS // _ZR, zcp, 0)
    pltpu.sync_copy(dst_hbm.at[wid], idx_v)
    pltpu.async_copy(_m_slice(0), b0, l0)
    plsc.subcore_barrier()

    def _step(i, j):
        jn = (j + 1) % 3

        @pl.when(i >= 2)
        def _():
            _drain_adds(bufs[jn], asem[jn], i - 2)

        @pl.when(i + 1 < _NSUP)
        def _():
            pltpu.async_copy(_m_slice(i + 1), bufs[jn], lsem[jn])

        pltpu.make_async_copy(_m_slice(i), bufs[j], lsem[j]).wait()
        _issue_adds(bufs[j], asem[j], i)

    def body(g, carry):
        for j in range(3):
            _step(3 * g + j, j)
        return carry

    lax.fori_loop(0, _NSUP // 3, body, 0)
    _step(_NSUP - 1, (_NSUP - 1) % 3)
    for i in range(_NSUP - 2, _NSUP):
        _drain_adds(bufs[i % 3], asem[i % 3], i)
    plsc.subcore_barrier()
    pltpu.sync_copy(
        acc_sh.at[pl.ds(s * _NPS, _NPS)], out_hbm.at[cc, pl.ds(s * _NPS, _NPS)]
    )


def _embed_body(x_ref, ewT_ref, eb_ref, w1hT_ref, b1_ref, h_ref, p_ref):
    h = jax.nn.silu(
        jnp.dot(x_ref[...], ewT_ref[...], preferred_element_type=jnp.float32)
        + eb_ref[...]
    )
    h_ref[...] = h
    p_ref[...] = (
        jnp.dot(h, w1hT_ref[...], preferred_element_type=jnp.float32) + b1_ref[...]
    )


_embed = pl.pallas_call(
    _embed_body,
    out_shape=[
        jax.ShapeDtypeStruct((_N2, 2 * _H), jnp.float32),
        jax.ShapeDtypeStruct((_N2, 2 * _H), jnp.float32),
    ],
)

_BE = 3200


def _msg_body(g_ref, ea_ref, w1eT_ref, w2T_ref, b2_ref, out_ref):
    m1 = jax.nn.silu(
        g_ref[...]
        + jnp.dot(ea_ref[...], w1eT_ref[...], preferred_element_type=jnp.float32)
    )
    out_ref[...] = jax.nn.silu(
        jnp.dot(m1, w2T_ref[...], preferred_element_type=jnp.float32) + b2_ref[...]
    )


_msg = pl.pallas_call(
    _msg_body,
    grid=(_E // 2 // _BE,),
    in_specs=[
        pl.BlockSpec((_BE, 2 * _H), lambda i: (i, 0)),
        pl.BlockSpec((_BE, 2 * _ED), lambda i: (i, 0)),
        pl.BlockSpec((2 * _ED, 2 * _H), lambda i: (0, 0)),
        pl.BlockSpec((2 * _H, 2 * _H), lambda i: (0, 0)),
        pl.BlockSpec((1, 2 * _H), lambda i: (0, 0)),
    ],
    out_specs=pl.BlockSpec((_BE, 2 * _H), lambda i: (i, 0)),
    out_shape=jax.ShapeDtypeStruct((_E // 2, 2 * _H), jnp.float32),
)


def _blockdiag2(w):
    z = jnp.zeros_like(w)
    return jnp.concatenate(
        [jnp.concatenate([w, z], axis=1), jnp.concatenate([z, w], axis=1)], axis=0
    )


def _gru_w2(W):
    WT = W.T
    return jnp.concatenate(
        [_blockdiag2(WT[:, k * _H : (k + 1) * _H]) for k in range(3)], axis=1
    )


def _gru_b2(b):
    return jnp.concatenate(
        [jnp.tile(b[k * _H : (k + 1) * _H], 2) for k in range(3)]
    )[None]


def _gru_core(h, a0_ref, a1_ref, wihT_ref, bih_ref, whhT_ref, bhh_ref):
    agg = a0_ref[...] + a1_ref[...]
    gi = (
        jnp.dot(agg, wihT_ref[...], preferred_element_type=jnp.float32)
        + bih_ref[...]
    )
    gh = jnp.dot(h, whhT_ref[...], preferred_element_type=jnp.float32) + bhh_ref[...]
    hp = 2 * _H
    r = jax.nn.sigmoid(gi[:, :hp] + gh[:, :hp])
    z = jax.nn.sigmoid(gi[:, hp : 2 * hp] + gh[:, hp : 2 * hp])
    n = jnp.tanh(gi[:, 2 * hp :] + r * gh[:, 2 * hp :])
    return (1.0 - z) * n + z * h


def _gru_body(
    h_ref, a0_ref, a1_ref, wihT_ref, bih_ref, whhT_ref, bhh_ref, w1hT_ref, b1_ref,
    hout_ref, pout_ref,
):
    hn = _gru_core(h_ref[...], a0_ref, a1_ref, wihT_ref, bih_ref, whhT_ref, bhh_ref)
    hout_ref[...] = hn
    pout_ref[...] = (
        jnp.dot(hn, w1hT_ref[...], preferred_element_type=jnp.float32) + b1_ref[...]
    )


_gru = pl.pallas_call(
    _gru_body,
    out_shape=[
        jax.ShapeDtypeStruct((_N2, 2 * _H), jnp.float32),
        jax.ShapeDtypeStruct((_N2, 2 * _H), jnp.float32),
    ],
)


def _gru_head_body(
    h_ref, a0_ref, a1_ref, wihT_ref, bih_ref, whhT_ref, bhh_ref, hw1T_ref, hb1_ref,
    hw2_ref, hb2_ref, out_ref,
):
    hn = _gru_core(h_ref[...], a0_ref, a1_ref, wihT_ref, bih_ref, whhT_ref, bhh_ref)
    sh = jax.nn.silu(
        jnp.dot(hn, hw1T_ref[...], preferred_element_type=jnp.float32) + hb1_ref[...]
    )
    v = jnp.sum(sh, axis=0, keepdims=True)
    total = jnp.sum(v * hw2_ref[...]) + _N * hb2_ref[0, 0]
    out_ref[...] = jnp.reshape(total, (1, 1))


_gru_head = pl.pallas_call(
    _gru_head_body,
    out_shape=jax.ShapeDtypeStruct((1, 1), jnp.float32),
)


def kernel(
    x, edge_index, edge_attr, embed_W, embed_b, msg_W1, msg_b1, msg_W2, msg_b2,
    gru_Wih, gru_bih, gru_Whh, gru_bhh, head_W1, head_b1, head_W2, head_b2,
):
    src = edge_index[0].reshape(_NW, _NCHUNK, _CHUNK)
    dst = edge_index[1].reshape(_NW, _NCHUNK, _CHUNK)
    h2, p2 = _embed(
        x.reshape(_N2, 2 * _D),
        _blockdiag2(embed_W.T),
        jnp.tile(embed_b, 2)[None],
        _blockdiag2(msg_W1[0, :, :_H].T),
        jnp.tile(msg_b1[0], 2)[None],
    )
    ea2 = edge_attr.reshape(_E // 2, 2 * _ED)
    out = None
    for l in range(_NL):
        g = _sc_gather(p2.reshape(_N, _H), src)
        m22 = _msg(
            g.reshape(_E // 2, 2 * _H),
            ea2,
            _blockdiag2(msg_W1[l, :, _H:].T),
            _blockdiag2(msg_W2[l].T),
            jnp.tile(msg_b2[l], 2)[None],
        )
        aggp = _sc_scatter(m22.reshape(_E, _H), dst)
        a2 = aggp.reshape(_NC, _N2, 2 * _H)
        if l < _NL - 1:
            h2, p2 = _gru(
                h2, a2[0], a2[1],
                _gru_w2(gru_Wih[l]), _gru_b2(gru_bih[l]),
                _gru_w2(gru_Whh[l]), _gru_b2(gru_bhh[l]),
                _blockdiag2(msg_W1[l + 1, :, :_H].T),
                jnp.tile(msg_b1[l + 1], 2)[None],
            )
        else:
            out = _gru_head(
                h2, a2[0], a2[1],
                _gru_w2(gru_Wih[l]), _gru_b2(gru_bih[l]),
                _gru_w2(gru_Whh[l]), _gru_b2(gru_bhh[l]),
                _blockdiag2(head_W1.T), jnp.tile(head_b1, 2)[None],
                jnp.tile(head_W2[0], 2)[None], head_b2.reshape(1, 1),
            )
    return out.reshape((1,))


# speedup vs baseline: 1.7088x; 1.0413x over previous
import functools

import jax
import jax.numpy as jnp
from jax import lax
from jax.experimental import pallas as pl
from jax.experimental.pallas import tpu as pltpu
from jax.experimental.pallas import tpu_sc as plsc

_N = 10000
_E = 320000
_D = 128
_ED = 16
_H = 64
_NL = 4
_N2 = _N // 2

_NC = 2
_NS = 16
_NW = _NC * _NS
_EPW = _E // _NW
_CHUNK = 80
_NCHUNK = _EPW // _CHUNK
_NPS = _N // _NS
_SB = 5
_SUP = _SB * _CHUNK
_NSUP = _EPW // _SUP

_mesh = plsc.VectorSubcoreMesh(
    core_axis_name="c", subcore_axis_name="s", num_cores=_NC, num_subcores=_NS
)


def _issue_gathers(p_hbm, idx_v, buf, sem, sup):
    for k in range(_SB):
        pltpu.async_copy(
            p_hbm.at[idx_v.at[sup * _SB + k]], buf.at[pl.ds(k * _CHUNK, _CHUNK)], sem
        )


def _drain_gathers(p_hbm, idx_v, buf, sem, sup):
    for k in range(_SB):
        pltpu.make_async_copy(
            p_hbm.at[idx_v.at[sup * _SB + k]], buf.at[pl.ds(k * _CHUNK, _CHUNK)], sem
        ).wait()


@functools.partial(
    pl.kernel,
    out_type=jax.ShapeDtypeStruct((_E, _H), jnp.float32),
    mesh=_mesh,
    scratch_types=[
        pltpu.VMEM((_NCHUNK, _CHUNK), jnp.int32),
        pltpu.VMEM((_SUP, _H), jnp.float32),
        pltpu.VMEM((_SUP, _H), jnp.float32),
        pltpu.VMEM((_SUP, _H), jnp.float32),
        pltpu.SemaphoreType.DMA,
        pltpu.SemaphoreType.DMA,
        pltpu.SemaphoreType.DMA,
        pltpu.SemaphoreType.DMA,
        pltpu.SemaphoreType.DMA,
        pltpu.SemaphoreType.DMA,
    ],
    compiler_params=pltpu.CompilerParams(use_tc_tiling_on_sc=False),
)
def _sc_gather(p_hbm, src_hbm, out_hbm, idx_v, b0, b1, b2, g0, g1, g2, s0, s1, s2):
    wid = lax.axis_index("c") * _NS + lax.axis_index("s")
    base = wid * _EPW
    bufs = (b0, b1, b2)
    gsem = (g0, g1, g2)
    ssem = (s0, s1, s2)
    pltpu.sync_copy(src_hbm.at[wid], idx_v)
    _issue_gathers(p_hbm, idx_v, b0, g0, 0)

    def _out_slice(sup):
        return out_hbm.at[pl.ds(base + sup * _SUP, _SUP)]

    def _step(i, j):
        jn = (j + 1) % 3

        @pl.when(i >= 2)
        def _():
            pltpu.make_async_copy(bufs[jn], _out_slice(i - 2), ssem[jn]).wait()

        @pl.when(i + 1 < _NSUP)
        def _():
            _issue_gathers(p_hbm, idx_v, bufs[jn], gsem[jn], i + 1)

        _drain_gathers(p_hbm, idx_v, bufs[j], gsem[j], i)
        pltpu.async_copy(bufs[j], _out_slice(i), ssem[j])

    def body(g, carry):
        for j in range(3):
            _step(3 * g + j, j)
        return carry

    lax.fori_loop(0, _NSUP // 3, body, 0)
    _step(_NSUP - 1, (_NSUP - 1) % 3)
    for i in range(_NSUP - 2, _NSUP):
        pltpu.make_async_copy(bufs[i % 3], _out_slice(i), ssem[i % 3]).wait()


@functools.partial(
    pl.kernel,
    out_type=jax.ShapeDtypeStruct((_NC, _N, _H), jnp.float32),
    mesh=_mesh,
    scratch_types=[
        pltpu.VMEM((_NCHUNK, _CHUNK), jnp.int32),
        pltpu.VMEM((_SUP, _H), jnp.float32),
        pltpu.VMEM((_SUP, _H), jnp.float32),
        pltpu.VMEM((_SUP, _H), jnp.float32),
        pltpu.VMEM((_NPS // 25, _H), jnp.float32),
        pltpu.VMEM_SHARED((_N, _H), jnp.float32),
        pltpu.SemaphoreType.DMA,
        pltpu.SemaphoreType.DMA,
        pltpu.SemaphoreType.DMA,
        pltpu.SemaphoreType.DMA,
        pltpu.SemaphoreType.DMA,
        pltpu.SemaphoreType.DMA,
    ],
    compiler_params=pltpu.CompilerParams(use_tc_tiling_on_sc=False),
)
def _sc_scatter(
    m_hbm, dst_hbm, out_hbm, idx_v, b0, b1, b2, zbuf, acc_sh, l0, l1, l2, a0, a1, a2
):
    cc = lax.axis_index("c")
    s = lax.axis_index("s")
    wid = cc * _NS + s
    base = wid * _EPW
    bufs = (b0, b1, b2)
    lsem = (l0, l1, l2)
    asem = (a0, a1, a2)

    def _m_slice(sup):
        return m_hbm.at[pl.ds(base + sup * _SUP, _SUP)]

    def _issue_adds(buf, sem, sup):
        for k in range(_SB):
            pltpu.async_copy(
                buf.at[pl.ds(k * _CHUNK, _CHUNK)],
                acc_sh.at[idx_v.at[sup * _SB + k]],
                sem,
                add=True,
            )

    def _drain_adds(buf, sem, sup):
        for k in range(_SB):
            pltpu.make_async_copy(
                buf.at[pl.ds(k * _CHUNK, _CHUNK)],
                acc_sh.at[idx_v.at[sup * _SB + k]],
                sem,
            ).wait()

    def zb(k, carry):
        zbuf[k // 4, pl.ds((k % 4) * 16, 16)] = jnp.zeros((16,), jnp.float32)
        return carry

    _ZR = _NPS // 25
    lax.fori_loop(0, _ZR * 4, zb, 0)

    def zcp(r, carry):
        pltpu.sync_copy(zbuf, acc_sh.at[pl.ds(s * _NPS + r * _ZR, _ZR)])
        return carry

    lax.fori_loop(0, _NPS // _ZR, zcp, 0)
    pltpu.sync_copy(dst_hbm.at[wid], idx_v)
    pltpu.async_copy(_m_slice(0), b0, l0)
    plsc.subcore_barrier()

    def _step(i, j):
        jn = (j + 1) % 3

        @pl.when(i >= 2)
        def _():
            _drain_adds(bufs[jn], asem[jn], i - 2)

        @pl.when(i + 1 < _NSUP)
        def _():
            pltpu.async_copy(_m_slice(i + 1), bufs[jn], lsem[jn])

        pltpu.make_async_copy(_m_slice(i), bufs[j], lsem[j]).wait()
        _issue_adds(bufs[j], asem[j], i)

    def body(g, carry):
        for j in range(3):
            _step(3 * g + j, j)
        return carry

    lax.fori_loop(0, _NSUP // 3, body, 0)
    _step(_NSUP - 1, (_NSUP - 1) % 3)
    for i in range(_NSUP - 2, _NSUP):
        _drain_adds(bufs[i % 3], asem[i % 3], i)
    plsc.subcore_barrier()
    pltpu.sync_copy(
        acc_sh.at[pl.ds(s * _NPS, _NPS)], out_hbm.at[cc, pl.ds(s * _NPS, _NPS)]
    )


def _embed_body(x_ref, ewT_ref, eb_ref, w1hT_ref, b1_ref, h_ref, p_ref):
    h = jax.nn.silu(
        jnp.dot(x_ref[...], ewT_ref[...], preferred_element_type=jnp.float32)
        + eb_ref[...]
    )
    h_ref[...] = h
    p_ref[...] = (
        jnp.dot(h, w1hT_ref[...], preferred_element_type=jnp.float32) + b1_ref[...]
    )


_embed = pl.pallas_call(
    _embed_body,
    out_shape=[
        jax.ShapeDtypeStruct((_N2, 2 * _H), jnp.float32),
        jax.ShapeDtypeStruct((_N2, 2 * _H), jnp.float32),
    ],
)

_BE = 6400


def _msg_body(g_ref, ea_ref, w1eT_ref, w2T_ref, b2_ref, out_ref):
    m1 = jax.nn.silu(
        g_ref[...]
        + jnp.dot(ea_ref[...], w1eT_ref[...], preferred_element_type=jnp.float32)
    )
    out_ref[...] = jax.nn.silu(
        jnp.dot(m1, w2T_ref[...], preferred_element_type=jnp.float32) + b2_ref[...]
    )


_msg = pl.pallas_call(
    _msg_body,
    grid=(_E // 2 // _BE,),
    in_specs=[
        pl.BlockSpec((_BE, 2 * _H), lambda i: (i, 0)),
        pl.BlockSpec((_BE, 2 * _ED), lambda i: (i, 0)),
        pl.BlockSpec((2 * _ED, 2 * _H), lambda i: (0, 0)),
        pl.BlockSpec((2 * _H, 2 * _H), lambda i: (0, 0)),
        pl.BlockSpec((1, 2 * _H), lambda i: (0, 0)),
    ],
    out_specs=pl.BlockSpec((_BE, 2 * _H), lambda i: (i, 0)),
    out_shape=jax.ShapeDtypeStruct((_E // 2, 2 * _H), jnp.float32),
)


def _blockdiag2(w):
    z = jnp.zeros_like(w)
    return jnp.concatenate(
        [jnp.concatenate([w, z], axis=1), jnp.concatenate([z, w], axis=1)], axis=0
    )


def _gru_w2(W):
    WT = W.T
    return jnp.concatenate(
        [_blockdiag2(WT[:, k * _H : (k + 1) * _H]) for k in range(3)], axis=1
    )


def _gru_b2(b):
    return jnp.concatenate(
        [jnp.tile(b[k * _H : (k + 1) * _H], 2) for k in range(3)]
    )[None]


def _gru_core(h, a0_ref, a1_ref, wihT_ref, bih_ref, whhT_ref, bhh_ref):
    agg = a0_ref[...] + a1_ref[...]
    gi = (
        jnp.dot(agg, wihT_ref[...], preferred_element_type=jnp.float32)
        + bih_ref[...]
    )
    gh = jnp.dot(h, whhT_ref[...], preferred_element_type=jnp.float32) + bhh_ref[...]
    hp = 2 * _H
    r = jax.nn.sigmoid(gi[:, :hp] + gh[:, :hp])
    z = jax.nn.sigmoid(gi[:, hp : 2 * hp] + gh[:, hp : 2 * hp])
    n = jnp.tanh(gi[:, 2 * hp :] + r * gh[:, 2 * hp :])
    return (1.0 - z) * n + z * h


def _gru_body(
    h_ref, a0_ref, a1_ref, wihT_ref, bih_ref, whhT_ref, bhh_ref, w1hT_ref, b1_ref,
    hout_ref, pout_ref,
):
    hn = _gru_core(h_ref[...], a0_ref, a1_ref, wihT_ref, bih_ref, whhT_ref, bhh_ref)
    hout_ref[...] = hn
    pout_ref[...] = (
        jnp.dot(hn, w1hT_ref[...], preferred_element_type=jnp.float32) + b1_ref[...]
    )


_gru = pl.pallas_call(
    _gru_body,
    out_shape=[
        jax.ShapeDtypeStruct((_N2, 2 * _H), jnp.float32),
        jax.ShapeDtypeStruct((_N2, 2 * _H), jnp.float32),
    ],
)


def _gru_head_body(
    h_ref, a0_ref, a1_ref, wihT_ref, bih_ref, whhT_ref, bhh_ref, hw1T_ref, hb1_ref,
    hw2_ref, hb2_ref, out_ref,
):
    hn = _gru_core(h_ref[...], a0_ref, a1_ref, wihT_ref, bih_ref, whhT_ref, bhh_ref)
    sh = jax.nn.silu(
        jnp.dot(hn, hw1T_ref[...], preferred_element_type=jnp.float32) + hb1_ref[...]
    )
    v = jnp.sum(sh, axis=0, keepdims=True)
    total = jnp.sum(v * hw2_ref[...]) + _N * hb2_ref[0, 0]
    out_ref[...] = jnp.reshape(total, (1, 1))


_gru_head = pl.pallas_call(
    _gru_head_body,
    out_shape=jax.ShapeDtypeStruct((1, 1), jnp.float32),
)


def kernel(
    x, edge_index, edge_attr, embed_W, embed_b, msg_W1, msg_b1, msg_W2, msg_b2,
    gru_Wih, gru_bih, gru_Whh, gru_bhh, head_W1, head_b1, head_W2, head_b2,
):
    src = edge_index[0].reshape(_NW, _NCHUNK, _CHUNK)
    dst = edge_index[1].reshape(_NW, _NCHUNK, _CHUNK)
    h2, p2 = _embed(
        x.reshape(_N2, 2 * _D),
        _blockdiag2(embed_W.T),
        jnp.tile(embed_b, 2)[None],
        _blockdiag2(msg_W1[0, :, :_H].T),
        jnp.tile(msg_b1[0], 2)[None],
    )
    ea2 = edge_attr.reshape(_E // 2, 2 * _ED)
    out = None
    for l in range(_NL):
        g = _sc_gather(p2.reshape(_N, _H), src)
        m22 = _msg(
            g.reshape(_E // 2, 2 * _H),
            ea2,
            _blockdiag2(msg_W1[l, :, _H:].T),
            _blockdiag2(msg_W2[l].T),
            jnp.tile(msg_b2[l], 2)[None],
        )
        aggp = _sc_scatter(m22.reshape(_E, _H), dst)
        a2 = aggp.reshape(_NC, _N2, 2 * _H)
        if l < _NL - 1:
            h2, p2 = _gru(
                h2, a2[0], a2[1],
                _gru_w2(gru_Wih[l]), _gru_b2(gru_bih[l]),
                _gru_w2(gru_Whh[l]), _gru_b2(gru_bhh[l]),
                _blockdiag2(msg_W1[l + 1, :, :_H].T),
                jnp.tile(msg_b1[l + 1], 2)[None],
            )
        else:
            out = _gru_head(
                h2, a2[0], a2[1],
                _gru_w2(gru_Wih[l]), _gru_b2(gru_bih[l]),
                _gru_w2(gru_Whh[l]), _gru_b2(gru_bhh[l]),
                _blockdiag2(head_W1.T), jnp.tile(head_b1, 2)[None],
                jnp.tile(head_W2[0], 2)[None], head_b2.reshape(1, 1),
            )
    return out.reshape((1,))
